# baseline (speedup 1.0000x reference)
import functools

import jax
import jax.numpy as jnp
from jax import lax
from jax.experimental import pallas as pl
from jax.experimental.pallas import tpu as pltpu
from jax.experimental.pallas import tpu_sc as plsc

N = 10000
E = 320000
CH = 128
NPAD = 10240
NC = 2
NS = 16
NW = NC * NS
EPW = E // NW
HALF = NPAD // 2
HPAD = HALF + 128
DUMP = HALF
EFIX = 5625
CAP = EFIX + 16
K = 125
EFC = EFIX // K
NST = 2 * EFC
RPH = HALF // NS

_sc_mesh = plsc.VectorSubcoreMesh(
    core_axis_name="c", subcore_axis_name="s", num_cores=NC, num_subcores=NS
)
_sc_params = pltpu.CompilerParams(needs_layout_passes=False)


@functools.partial(
    pl.kernel,
    out_type=[
        jax.ShapeDtypeStruct((NW, NPAD), jnp.float32),
        jax.ShapeDtypeStruct((NW, 2, 2, CAP), jnp.int32),
    ],
    mesh=_sc_mesh,
    compiler_params=_sc_params,
    scratch_types=[
        pltpu.VMEM((2, EPW), jnp.int32),
        pltpu.VMEM((NPAD,), jnp.float32),
        pltpu.VMEM((CAP,), jnp.int32),
        pltpu.VMEM((CAP,), jnp.int32),
        pltpu.VMEM((CAP,), jnp.int32),
        pltpu.VMEM((CAP,), jnp.int32),
    ],
)
def _deg_kernel(
    sd_hbm, hist_hbm, plist_hbm,
    sd_v, hist_v, asrc_v, adst_v, bsrc_v, bdst_v,
):
    wid = lax.axis_index("c") * NS + lax.axis_index("s")
    pltpu.sync_copy(sd_hbm.at[wid], sd_v)

    zeros16 = jnp.zeros((16,), jnp.float32)

    def zbody(i, carry):
        hist_v[pl.ds(i * 16, 16)] = zeros16
        return carry

    lax.fori_loop(0, NPAD // 16, zbody, 0)

    ones16 = jnp.ones((16,), jnp.float32)

    def hbody(g, carry):
        off_a, off_b = carry
        src16 = sd_v[0, pl.ds(g * 16, 16)]
        dst16 = sd_v[1, pl.ds(g * 16, 16)]
        plsc.addupdate_scatter(hist_v, [dst16], ones16)
        mask = dst16 < HALF
        nmask = jnp.logical_not(mask)
        m32 = mask.astype(jnp.int32)
        nm32 = nmask.astype(jnp.int32)
        pos_a = off_a + plsc.cumsum(m32) - m32
        pos_b = off_b + plsc.cumsum(nm32) - nm32
        plsc.store_scatter(asrc_v, [pos_a], src16, mask=mask)
        plsc.store_scatter(adst_v, [pos_a], dst16, mask=mask)
        rel_b = dst16 - HALF
        plsc.store_scatter(bsrc_v, [pos_b], src16, mask=nmask)
        plsc.store_scatter(bdst_v, [pos_b], rel_b, mask=nmask)
        cnt_a = jnp.sum(m32)
        return off_a + cnt_a, off_b + (16 - cnt_a)

    off_a, off_b = lax.fori_loop(
        0, EPW // 16, hbody, (jnp.int32(0), jnp.int32(0))
    )

    off_a = jnp.minimum(off_a, EFIX)
    off_b = jnp.minimum(off_b, EFIX)
    zeros16i = jnp.zeros((16,), jnp.int32)
    ii16 = jax.lax.iota(jnp.int32, 16)

    def pad_list(off, src_ref, dst_ref):
        def pbody(i, carry):
            pos = off + 16 * i + ii16
            plsc.store_scatter(src_ref, [pos], zeros16i)
            plsc.store_scatter(dst_ref, [pos], DUMP + (pos % 128))
            return carry

        lax.fori_loop(0, (EFIX - off + 15) // 16, pbody, 0)

    pad_list(off_a, asrc_v, adst_v)
    pad_list(off_b, bsrc_v, bdst_v)

    pltpu.sync_copy(asrc_v, plist_hbm.at[wid, 0, 0])
    pltpu.sync_copy(adst_v, plist_hbm.at[wid, 0, 1])
    pltpu.sync_copy(bsrc_v, plist_hbm.at[wid, 1, 0])
    pltpu.sync_copy(bdst_v, plist_hbm.at[wid, 1, 1])


def _mm_body(x_ref, w_ref, h_ref, y_ref, dis_ref):
    deg = jnp.sum(h_ref[...], axis=0) + 1.0
    dis = lax.rsqrt(deg)
    z = jnp.dot(x_ref[...], w_ref[...], preferred_element_type=jnp.float32)
    y_ref[...] = z * dis[:, None]
    dis_ref[...] = dis[:, None]


_MM_BLK = 1024
_mm_call = pl.pallas_call(
    _mm_body,
    grid=(NPAD // _MM_BLK,),
    in_specs=[
        pl.BlockSpec((_MM_BLK, CH), lambda i: (i, 0)),
        pl.BlockSpec((CH, CH), lambda i: (0, 0)),
        pl.BlockSpec((NW, _MM_BLK), lambda i: (0, i)),
    ],
    out_specs=[
        pl.BlockSpec((_MM_BLK, CH), lambda i: (i, 0)),
        pl.BlockSpec((_MM_BLK, 1), lambda i: (i, 0)),
    ],
    out_shape=[
        jax.ShapeDtypeStruct((NPAD, CH), jnp.float32),
        jax.ShapeDtypeStruct((NPAD, 1), jnp.float32),
    ],
)


@functools.partial(
    pl.kernel,
    out_type=jax.ShapeDtypeStruct((NC, HALF, CH), jnp.float32),
    mesh=_sc_mesh,
    compiler_params=_sc_params,
    scratch_types=[
        pltpu.VMEM((NST, K), jnp.int32),
        pltpu.VMEM((NST, K), jnp.int32),
        pltpu.VMEM((K, CH), jnp.float32),
        pltpu.VMEM((80, CH), jnp.float32),
        pltpu.VMEM((RPH,), jnp.float32),
        pltpu.VMEM((CH,), jnp.float32),
        pltpu.VMEM_SHARED((HPAD, CH), jnp.float32),
    ],
)
def _agg_kernel(
    y_hbm, psrc_hbm, pdst_hbm, dis_hbm, b_hbm, out_hbm,
    lsrc_v, ldst_v, rows_v, dbuf_v, dis_v, b_v, acc,
):
    core = lax.axis_index("c")
    sub = lax.axis_index("s")
    base = sub * RPH

    pltpu.sync_copy(
        y_hbm.at[pl.ds(core * HALF + base, RPH)], acc.at[pl.ds(base, RPH)]
    )

    @pl.when(sub == 0)
    def _():
        pltpu.sync_copy(
            y_hbm.at[pl.ds(0, HPAD - HALF)], acc.at[pl.ds(HALF, HPAD - HALF)]
        )

    pltpu.sync_copy(psrc_hbm.at[2 * sub, core], lsrc_v.at[pl.ds(0, EFC)])
    pltpu.sync_copy(psrc_hbm.at[2 * sub + 1, core], lsrc_v.at[pl.ds(EFC, EFC)])
    pltpu.sync_copy(pdst_hbm.at[2 * sub, core], ldst_v.at[pl.ds(0, EFC)])
    pltpu.sync_copy(pdst_hbm.at[2 * sub + 1, core], ldst_v.at[pl.ds(EFC, EFC)])
    plsc.subcore_barrier()

    def body(t, carry):
        pltpu.sync_copy(y_hbm.at[lsrc_v.at[t]], rows_v)
        pltpu.sync_copy(rows_v, acc.at[ldst_v.at[t]], add=True)
        return carry

    lax.fori_loop(0, NST, body, 0)

    plsc.subcore_barrier()

    pltpu.sync_copy(dis_hbm.at[pl.ds(core * HALF + base, RPH)], dis_v)
    pltpu.sync_copy(b_hbm, b_v)

    def drain(q, carry):
        pltpu.sync_copy(acc.at[pl.ds(base + 80 * q, 80)], dbuf_v)

        def row(r, carry2):
            ridx = jnp.zeros((16,), jnp.int32) + (80 * q + r)
            d = plsc.load_gather(dis_v, [ridx])
            for u in range(CH // 16):
                cs = pl.ds(16 * u, 16)
                dbuf_v[r, cs] = dbuf_v[r, cs] * d + b_v[cs]
            return carry2

        lax.fori_loop(0, 80, row, 0)
        pltpu.sync_copy(dbuf_v, out_hbm.at[core, pl.ds(base + 80 * q, 80)])
        return carry

    lax.fori_loop(0, RPH // 80, drain, 0)


def kernel(x, edge_index, W, b):
    src = edge_index[0].astype(jnp.int32)
    dst = edge_index[1].astype(jnp.int32)
    sd = jnp.stack([src.reshape(NW, EPW), dst.reshape(NW, EPW)], axis=1)
    hist, plist = _deg_kernel(sd)
    x_pad = jnp.pad(x, ((0, NPAD - N), (0, 0)))
    yp, dis = _mm_call(x_pad, W, hist)
    psrc = plist[:, :, 0, :EFIX].reshape(NW, 2, EFC, K)
    pdst = plist[:, :, 1, :EFIX].reshape(NW, 2, EFC, K)
    parts = _agg_kernel(yp, psrc, pdst, dis.reshape(NPAD), b)
    return jnp.concatenate([parts[0], parts[1, : N - HALF]], axis=0)


# speedup vs baseline: 1.0008x; 1.0008x over previous
import functools

import jax
import jax.numpy as jnp
from jax import lax
from jax.experimental import pallas as pl
from jax.experimental.pallas import tpu as pltpu
from jax.experimental.pallas import tpu_sc as plsc

N = 10000
E = 320000
CH = 128
NPAD = 10240
NC = 2
NS = 16
NW = NC * NS
EPW = E // NW
HALF = NPAD // 2
HPAD = HALF + 128
DUMP = HALF
EFIX = 5625
CAP = EFIX + 16
K = 125
EFC = EFIX // K
NST = 2 * EFC
RPH = HALF // NS

_sc_mesh = plsc.VectorSubcoreMesh(
    core_axis_name="c", subcore_axis_name="s", num_cores=NC, num_subcores=NS
)
_sc_params = pltpu.CompilerParams(needs_layout_passes=False)


@functools.partial(
    pl.kernel,
    out_type=[
        jax.ShapeDtypeStruct((NW, NPAD), jnp.float32),
        jax.ShapeDtypeStruct((NW, 2, 2, CAP), jnp.int32),
    ],
    mesh=_sc_mesh,
    compiler_params=_sc_params,
    scratch_types=[
        pltpu.VMEM((2, EPW), jnp.int32),
        pltpu.VMEM((NPAD,), jnp.float32),
        pltpu.VMEM((CAP,), jnp.int32),
        pltpu.VMEM((CAP,), jnp.int32),
        pltpu.VMEM((CAP,), jnp.int32),
        pltpu.VMEM((CAP,), jnp.int32),
    ],
)
def _deg_kernel(
    sd_hbm, hist_hbm, plist_hbm,
    sd_v, hist_v, asrc_v, adst_v, bsrc_v, bdst_v,
):
    wid = lax.axis_index("c") * NS + lax.axis_index("s")
    pltpu.sync_copy(sd_hbm.at[wid], sd_v)

    zeros16 = jnp.zeros((16,), jnp.float32)

    def zbody(i, carry):
        hist_v[pl.ds(i * 16, 16)] = zeros16
        return carry

    lax.fori_loop(0, NPAD // 16, zbody, 0)

    ones16 = jnp.ones((16,), jnp.float32)

    def hbody(g, carry):
        off_a, off_b = carry
        src16 = sd_v[0, pl.ds(g * 16, 16)]
        dst16 = sd_v[1, pl.ds(g * 16, 16)]
        plsc.addupdate_scatter(hist_v, [dst16], ones16)
        mask = dst16 < HALF
        nmask = jnp.logical_not(mask)
        m32 = mask.astype(jnp.int32)
        nm32 = nmask.astype(jnp.int32)
        pos_a = off_a + plsc.cumsum(m32) - m32
        pos_b = off_b + plsc.cumsum(nm32) - nm32
        plsc.store_scatter(asrc_v, [pos_a], src16, mask=mask)
        plsc.store_scatter(adst_v, [pos_a], dst16, mask=mask)
        rel_b = dst16 - HALF
        plsc.store_scatter(bsrc_v, [pos_b], src16, mask=nmask)
        plsc.store_scatter(bdst_v, [pos_b], rel_b, mask=nmask)
        cnt_a = jnp.sum(m32)
        return off_a + cnt_a, off_b + (16 - cnt_a)

    off_a, off_b = lax.fori_loop(
        0, EPW // 16, hbody, (jnp.int32(0), jnp.int32(0))
    )

    off_a = jnp.minimum(off_a, EFIX)
    off_b = jnp.minimum(off_b, EFIX)
    zeros16i = jnp.zeros((16,), jnp.int32)
    ii16 = jax.lax.iota(jnp.int32, 16)

    def pad_list(off, src_ref, dst_ref):
        def pbody(i, carry):
            pos = off + 16 * i + ii16
            pm = pos < EFIX
            plsc.store_scatter(src_ref, [pos], zeros16i, mask=pm)
            plsc.store_scatter(dst_ref, [pos], DUMP + (pos % 128), mask=pm)
            return carry

        lax.fori_loop(0, (EFIX - 4300 + 15) // 16, pbody, 0)

    pad_list(off_a, asrc_v, adst_v)
    pad_list(off_b, bsrc_v, bdst_v)

    pltpu.sync_copy(asrc_v, plist_hbm.at[wid, 0, 0])
    pltpu.sync_copy(adst_v, plist_hbm.at[wid, 0, 1])
    pltpu.sync_copy(bsrc_v, plist_hbm.at[wid, 1, 0])
    pltpu.sync_copy(bdst_v, plist_hbm.at[wid, 1, 1])


def _mm_body(x_ref, w_ref, h_ref, y_ref, dis_ref):
    deg = jnp.sum(h_ref[...], axis=0) + 1.0
    dis = lax.rsqrt(deg)
    z = jnp.dot(x_ref[...], w_ref[...], preferred_element_type=jnp.float32)
    y_ref[...] = z * dis[:, None]
    dis_ref[...] = dis[:, None]


_MM_BLK = 1024
_mm_call = pl.pallas_call(
    _mm_body,
    grid=(NPAD // _MM_BLK,),
    in_specs=[
        pl.BlockSpec((_MM_BLK, CH), lambda i: (i, 0)),
        pl.BlockSpec((CH, CH), lambda i: (0, 0)),
        pl.BlockSpec((NW, _MM_BLK), lambda i: (0, i)),
    ],
    out_specs=[
        pl.BlockSpec((_MM_BLK, CH), lambda i: (i, 0)),
        pl.BlockSpec((_MM_BLK, 1), lambda i: (i, 0)),
    ],
    out_shape=[
        jax.ShapeDtypeStruct((NPAD, CH), jnp.float32),
        jax.ShapeDtypeStruct((NPAD, 1), jnp.float32),
    ],
)


@functools.partial(
    pl.kernel,
    out_type=jax.ShapeDtypeStruct((NC, HALF, CH), jnp.float32),
    mesh=_sc_mesh,
    compiler_params=_sc_params,
    scratch_types=[
        pltpu.VMEM((NST, K), jnp.int32),
        pltpu.VMEM((NST, K), jnp.int32),
        pltpu.VMEM((K, CH), jnp.float32),
        pltpu.VMEM((80, CH), jnp.float32),
        pltpu.VMEM((RPH,), jnp.float32),
        pltpu.VMEM((CH,), jnp.float32),
        pltpu.VMEM_SHARED((HPAD, CH), jnp.float32),
    ],
)
def _agg_kernel(
    y_hbm, psrc_hbm, pdst_hbm, dis_hbm, b_hbm, out_hbm,
    lsrc_v, ldst_v, rows_v, dbuf_v, dis_v, b_v, acc,
):
    core = lax.axis_index("c")
    sub = lax.axis_index("s")
    base = sub * RPH

    pltpu.sync_copy(
        y_hbm.at[pl.ds(core * HALF + base, RPH)], acc.at[pl.ds(base, RPH)]
    )

    @pl.when(sub == 0)
    def _():
        pltpu.sync_copy(
            y_hbm.at[pl.ds(0, HPAD - HALF)], acc.at[pl.ds(HALF, HPAD - HALF)]
        )

    pltpu.sync_copy(psrc_hbm.at[2 * sub, core], lsrc_v.at[pl.ds(0, EFC)])
    pltpu.sync_copy(psrc_hbm.at[2 * sub + 1, core], lsrc_v.at[pl.ds(EFC, EFC)])
    pltpu.sync_copy(pdst_hbm.at[2 * sub, core], ldst_v.at[pl.ds(0, EFC)])
    pltpu.sync_copy(pdst_hbm.at[2 * sub + 1, core], ldst_v.at[pl.ds(EFC, EFC)])
    plsc.subcore_barrier()

    def body(t, carry):
        pltpu.sync_copy(y_hbm.at[lsrc_v.at[t]], rows_v)
        pltpu.sync_copy(rows_v, acc.at[ldst_v.at[t]], add=True)
        return carry

    lax.fori_loop(0, NST, body, 0)

    plsc.subcore_barrier()

    pltpu.sync_copy(dis_hbm.at[pl.ds(core * HALF + base, RPH)], dis_v)
    pltpu.sync_copy(b_hbm, b_v)

    def drain(q, carry):
        pltpu.sync_copy(acc.at[pl.ds(base + 80 * q, 80)], dbuf_v)

        def row(r, carry2):
            ridx = jnp.zeros((16,), jnp.int32) + (80 * q + r)
            d = plsc.load_gather(dis_v, [ridx])
            for u in range(CH // 16):
                cs = pl.ds(16 * u, 16)
                dbuf_v[r, cs] = dbuf_v[r, cs] * d + b_v[cs]
            return carry2

        lax.fori_loop(0, 80, row, 0)
        pltpu.sync_copy(dbuf_v, out_hbm.at[core, pl.ds(base + 80 * q, 80)])
        return carry

    lax.fori_loop(0, RPH // 80, drain, 0)


def kernel(x, edge_index, W, b):
    src = edge_index[0].astype(jnp.int32)
    dst = edge_index[1].astype(jnp.int32)
    sd = jnp.stack([src.reshape(NW, EPW), dst.reshape(NW, EPW)], axis=1)
    hist, plist = _deg_kernel(sd)
    x_pad = jnp.pad(x, ((0, NPAD - N), (0, 0)))
    yp, dis = _mm_call(x_pad, W, hist)
    psrc = plist[:, :, 0, :EFIX].reshape(NW, 2, EFC, K)
    pdst = plist[:, :, 1, :EFIX].reshape(NW, 2, EFC, K)
    parts = _agg_kernel(yp, psrc, pdst, dis.reshape(NPAD), b)
    return jnp.concatenate([parts[0], parts[1, : N - HALF]], axis=0)


# speedup vs baseline: 5.8209x; 5.8161x over previous
import functools

import jax
import jax.numpy as jnp
from jax import lax
from jax.experimental import pallas as pl
from jax.experimental.pallas import tpu as pltpu
from jax.experimental.pallas import tpu_sc as plsc

N = 10000
E = 320000
CH = 128
NPAD = 10240
NC = 2
NS = 16
NW = NC * NS
EPW = E // NW
HALF = NPAD // 2
HPAD = HALF + 128
DUMP = HALF
EFIX = 5625
CAP = EFIX + 16
K = 125
EFC = EFIX // K
NST = 2 * EFC
RPH = HALF // NS

_sc_mesh = plsc.VectorSubcoreMesh(
    core_axis_name="c", subcore_axis_name="s", num_cores=NC, num_subcores=NS
)
_sc_params = pltpu.CompilerParams(needs_layout_passes=False)


@functools.partial(
    pl.kernel,
    out_type=[
        jax.ShapeDtypeStruct((NW, NPAD), jnp.float32),
        jax.ShapeDtypeStruct((NW, 2, 2, CAP), jnp.int32),
    ],
    mesh=_sc_mesh,
    compiler_params=_sc_params,
    scratch_types=[
        pltpu.VMEM((2, EPW), jnp.int32),
        pltpu.VMEM((NPAD,), jnp.float32),
        pltpu.VMEM((CAP,), jnp.int32),
        pltpu.VMEM((CAP,), jnp.int32),
        pltpu.VMEM((CAP,), jnp.int32),
        pltpu.VMEM((CAP,), jnp.int32),
    ],
)
def _deg_kernel(
    sd_hbm, hist_hbm, plist_hbm,
    sd_v, hist_v, asrc_v, adst_v, bsrc_v, bdst_v,
):
    wid = lax.axis_index("c") * NS + lax.axis_index("s")
    pltpu.sync_copy(sd_hbm.at[wid], sd_v)

    zeros16 = jnp.zeros((16,), jnp.float32)

    def zbody(i, carry):
        hist_v[pl.ds(i * 16, 16)] = zeros16
        return carry

    lax.fori_loop(0, NPAD // 16, zbody, 0)

    ones16 = jnp.ones((16,), jnp.float32)

    def hbody(g, carry):
        off_a, off_b = carry
        src16 = sd_v[0, pl.ds(g * 16, 16)]
        dst16 = sd_v[1, pl.ds(g * 16, 16)]
        plsc.addupdate_scatter(hist_v, [dst16], ones16)
        mask = dst16 < HALF
        nmask = jnp.logical_not(mask)
        m32 = mask.astype(jnp.int32)
        nm32 = nmask.astype(jnp.int32)
        pos_a = off_a + plsc.cumsum(m32) - m32
        pos_b = off_b + plsc.cumsum(nm32) - nm32
        plsc.store_scatter(asrc_v, [pos_a], src16, mask=mask)
        plsc.store_scatter(adst_v, [pos_a], dst16, mask=mask)
        rel_b = dst16 - HALF
        plsc.store_scatter(bsrc_v, [pos_b], src16, mask=nmask)
        plsc.store_scatter(bdst_v, [pos_b], rel_b, mask=nmask)
        cnt_a = jnp.sum(m32)
        return off_a + cnt_a, off_b + (16 - cnt_a)

    off_a, off_b = lax.fori_loop(
        0, EPW // 16, hbody, (jnp.int32(0), jnp.int32(0))
    )

    off_a = jnp.minimum(off_a, EFIX)
    off_b = jnp.minimum(off_b, EFIX)
    zeros16i = jnp.zeros((16,), jnp.int32)
    ii16 = jax.lax.iota(jnp.int32, 16)

    def pad_list(off, src_ref, dst_ref):
        def pbody(i, carry):
            pos = off + 16 * i + ii16
            pm = pos < EFIX
            plsc.store_scatter(src_ref, [pos], pos % 4096, mask=pm)
            plsc.store_scatter(dst_ref, [pos], DUMP + (pos % 128), mask=pm)
            return carry

        lax.fori_loop(0, (EFIX - 4300 + 15) // 16, pbody, 0)

    pad_list(off_a, asrc_v, adst_v)
    pad_list(off_b, bsrc_v, bdst_v)

    pltpu.sync_copy(asrc_v, plist_hbm.at[wid, 0, 0])
    pltpu.sync_copy(adst_v, plist_hbm.at[wid, 0, 1])
    pltpu.sync_copy(bsrc_v, plist_hbm.at[wid, 1, 0])
    pltpu.sync_copy(bdst_v, plist_hbm.at[wid, 1, 1])


def _mm_body(x_ref, w_ref, h_ref, y_ref, dis_ref):
    deg = jnp.sum(h_ref[...], axis=0) + 1.0
    dis = lax.rsqrt(deg)
    z = jnp.dot(x_ref[...], w_ref[...], preferred_element_type=jnp.float32)
    y_ref[...] = z * dis[:, None]
    dis_ref[...] = dis[:, None]


_MM_BLK = 1024
_mm_call = pl.pallas_call(
    _mm_body,
    grid=(NPAD // _MM_BLK,),
    in_specs=[
        pl.BlockSpec((_MM_BLK, CH), lambda i: (i, 0)),
        pl.BlockSpec((CH, CH), lambda i: (0, 0)),
        pl.BlockSpec((NW, _MM_BLK), lambda i: (0, i)),
    ],
    out_specs=[
        pl.BlockSpec((_MM_BLK, CH), lambda i: (i, 0)),
        pl.BlockSpec((_MM_BLK, 1), lambda i: (i, 0)),
    ],
    out_shape=[
        jax.ShapeDtypeStruct((NPAD, CH), jnp.float32),
        jax.ShapeDtypeStruct((NPAD, 1), jnp.float32),
    ],
)


@functools.partial(
    pl.kernel,
    out_type=jax.ShapeDtypeStruct((NC, HALF, CH), jnp.float32),
    mesh=_sc_mesh,
    compiler_params=_sc_params,
    scratch_types=[
        pltpu.VMEM((NST, K), jnp.int32),
        pltpu.VMEM((NST, K), jnp.int32),
        pltpu.VMEM((K, CH), jnp.float32),
        pltpu.VMEM((80, CH), jnp.float32),
        pltpu.VMEM((RPH,), jnp.float32),
        pltpu.VMEM((CH,), jnp.float32),
        pltpu.VMEM_SHARED((HPAD, CH), jnp.float32),
    ],
)
def _agg_kernel(
    y_hbm, psrc_hbm, pdst_hbm, dis_hbm, b_hbm, out_hbm,
    lsrc_v, ldst_v, rows_v, dbuf_v, dis_v, b_v, acc,
):
    core = lax.axis_index("c")
    sub = lax.axis_index("s")
    base = sub * RPH

    pltpu.sync_copy(
        y_hbm.at[pl.ds(core * HALF + base, RPH)], acc.at[pl.ds(base, RPH)]
    )

    @pl.when(sub == 0)
    def _():
        pltpu.sync_copy(
            y_hbm.at[pl.ds(0, HPAD - HALF)], acc.at[pl.ds(HALF, HPAD - HALF)]
        )

    pltpu.sync_copy(psrc_hbm.at[2 * sub, core], lsrc_v.at[pl.ds(0, EFC)])
    pltpu.sync_copy(psrc_hbm.at[2 * sub + 1, core], lsrc_v.at[pl.ds(EFC, EFC)])
    pltpu.sync_copy(pdst_hbm.at[2 * sub, core], ldst_v.at[pl.ds(0, EFC)])
    pltpu.sync_copy(pdst_hbm.at[2 * sub + 1, core], ldst_v.at[pl.ds(EFC, EFC)])
    plsc.subcore_barrier()

    def body(t, carry):
        pltpu.sync_copy(y_hbm.at[lsrc_v.at[t]], rows_v)
        pltpu.sync_copy(rows_v, acc.at[ldst_v.at[t]], add=True)
        return carry

    lax.fori_loop(0, NST, body, 0)

    plsc.subcore_barrier()

    pltpu.sync_copy(dis_hbm.at[pl.ds(core * HALF + base, RPH)], dis_v)
    pltpu.sync_copy(b_hbm, b_v)

    def drain(q, carry):
        pltpu.sync_copy(acc.at[pl.ds(base + 80 * q, 80)], dbuf_v)

        def row(r, carry2):
            ridx = jnp.zeros((16,), jnp.int32) + (80 * q + r)
            d = plsc.load_gather(dis_v, [ridx])
            for u in range(CH // 16):
                cs = pl.ds(16 * u, 16)
                dbuf_v[r, cs] = dbuf_v[r, cs] * d + b_v[cs]
            return carry2

        lax.fori_loop(0, 80, row, 0)
        pltpu.sync_copy(dbuf_v, out_hbm.at[core, pl.ds(base + 80 * q, 80)])
        return carry

    lax.fori_loop(0, RPH // 80, drain, 0)


def kernel(x, edge_index, W, b):
    src = edge_index[0].astype(jnp.int32)
    dst = edge_index[1].astype(jnp.int32)
    sd = jnp.stack([src.reshape(NW, EPW), dst.reshape(NW, EPW)], axis=1)
    hist, plist = _deg_kernel(sd)
    x_pad = jnp.pad(x, ((0, NPAD - N), (0, 0)))
    yp, dis = _mm_call(x_pad, W, hist)
    psrc = plist[:, :, 0, :EFIX].reshape(NW, 2, EFC, K)
    pdst = plist[:, :, 1, :EFIX].reshape(NW, 2, EFC, K)
    parts = _agg_kernel(yp, psrc, pdst, dis.reshape(NPAD), b)
    return jnp.concatenate([parts[0], parts[1, : N - HALF]], axis=0)


# speedup vs baseline: 7.1754x; 1.2327x over previous
import functools

import jax
import jax.numpy as jnp
from jax import lax
from jax.experimental import pallas as pl
from jax.experimental.pallas import tpu as pltpu
from jax.experimental.pallas import tpu_sc as plsc

N = 10000
E = 320000
CH = 128
NPAD = 10240
NC = 2
NS = 16
NW = NC * NS
EPW = E // NW
K = 125
NCHUNK = EPW // K
RPT = NPAD // NS

_sc_mesh = plsc.VectorSubcoreMesh(
    core_axis_name="c", subcore_axis_name="s", num_cores=NC, num_subcores=NS
)
_sc_params = pltpu.CompilerParams(needs_layout_passes=False)


@functools.partial(
    pl.kernel,
    out_type=jax.ShapeDtypeStruct((NW, NPAD), jnp.float32),
    mesh=_sc_mesh,
    compiler_params=_sc_params,
    scratch_types=[
        pltpu.VMEM((EPW,), jnp.int32),
        pltpu.VMEM((NPAD,), jnp.float32),
    ],
)
def _deg_kernel(dst_hbm, out_hbm, idx_v, hist_v):
    wid = lax.axis_index("c") * NS + lax.axis_index("s")
    pltpu.sync_copy(dst_hbm.at[pl.ds(wid * EPW, EPW)], idx_v)

    zeros16 = jnp.zeros((16,), jnp.float32)

    def zbody(i, carry):
        hist_v[pl.ds(i * 16, 16)] = zeros16
        return carry

    lax.fori_loop(0, NPAD // 16, zbody, 0)

    ones16 = jnp.ones((16,), jnp.float32)

    def hbody(g, carry):
        idx = idx_v[pl.ds(g * 16, 16)]
        plsc.addupdate_scatter(hist_v, [idx], ones16)
        return carry

    lax.fori_loop(0, EPW // 16, hbody, 0)

    pltpu.sync_copy(hist_v, out_hbm.at[wid])


def _mm_body(x_ref, w_ref, h_ref, y_ref, dis_ref):
    deg = jnp.sum(h_ref[...], axis=0) + 1.0
    dis = lax.rsqrt(deg)
    z = jnp.dot(x_ref[...], w_ref[...], preferred_element_type=jnp.float32)
    y_ref[...] = z * dis[:, None]
    dis_ref[...] = dis[:, None]


_MM_BLK = 1024
_mm_call = pl.pallas_call(
    _mm_body,
    grid=(NPAD // _MM_BLK,),
    in_specs=[
        pl.BlockSpec((_MM_BLK, CH), lambda i: (i, 0)),
        pl.BlockSpec((CH, CH), lambda i: (0, 0)),
        pl.BlockSpec((NW, _MM_BLK), lambda i: (0, i)),
    ],
    out_specs=[
        pl.BlockSpec((_MM_BLK, CH), lambda i: (i, 0)),
        pl.BlockSpec((_MM_BLK, 1), lambda i: (i, 0)),
    ],
    out_shape=[
        jax.ShapeDtypeStruct((NPAD, CH), jnp.float32),
        jax.ShapeDtypeStruct((NPAD, 1), jnp.float32),
    ],
)


@functools.partial(
    pl.kernel,
    out_type=jax.ShapeDtypeStruct((NC, NPAD, CH), jnp.float32),
    mesh=_sc_mesh,
    compiler_params=_sc_params,
    scratch_types=[
        pltpu.VMEM((NCHUNK, K), jnp.int32),
        pltpu.VMEM((NCHUNK, K), jnp.int32),
        pltpu.VMEM((K, CH), jnp.float32),
        pltpu.VMEM_SHARED((NPAD, CH), jnp.float32),
    ],
)
def _agg_kernel(y_hbm, z_hbm, src_hbm, dst_hbm, out_hbm, src_v, dst_v, rows_v, acc):
    core = lax.axis_index("c")
    sub = lax.axis_index("s")
    wid = core * NS + sub
    sl = pl.ds(sub * RPT, RPT)

    @pl.when(core == 0)
    def _():
        pltpu.sync_copy(y_hbm.at[sl], acc.at[sl])

    @pl.when(core == 1)
    def _():
        pltpu.sync_copy(z_hbm.at[sl], acc.at[sl])

    pltpu.sync_copy(src_hbm.at[wid], src_v)
    pltpu.sync_copy(dst_hbm.at[wid], dst_v)
    plsc.subcore_barrier()

    def body(j, carry):
        pltpu.sync_copy(y_hbm.at[src_v.at[j]], rows_v)
        pltpu.sync_copy(rows_v, acc.at[dst_v.at[j]], add=True)
        return carry

    lax.fori_loop(0, NCHUNK, body, 0)

    plsc.subcore_barrier()
    pltpu.sync_copy(acc.at[sl], out_hbm.at[core].at[sl])


def _fin_body(p_ref, dis_ref, b_ref, o_ref):
    s = p_ref[0] + p_ref[1]
    o_ref[...] = s * dis_ref[...] + b_ref[...]


_FIN_BLK = 1000
_fin_call = pl.pallas_call(
    _fin_body,
    grid=(N // _FIN_BLK,),
    in_specs=[
        pl.BlockSpec((NC, _FIN_BLK, CH), lambda i: (0, i, 0)),
        pl.BlockSpec((_FIN_BLK, 1), lambda i: (i, 0)),
        pl.BlockSpec((1, CH), lambda i: (0, 0)),
    ],
    out_specs=pl.BlockSpec((_FIN_BLK, CH), lambda i: (i, 0)),
    out_shape=jax.ShapeDtypeStruct((N, CH), jnp.float32),
)


def kernel(x, edge_index, W, b):
    src = edge_index[0].astype(jnp.int32)
    dst = edge_index[1].astype(jnp.int32)
    hist = _deg_kernel(dst)
    x_pad = jnp.pad(x, ((0, NPAD - N), (0, 0)))
    yp, dis = _mm_call(x_pad, W, hist)
    zeros = jnp.zeros((NPAD, CH), jnp.float32)
    parts = _agg_kernel(
        yp, zeros, src.reshape(NW, NCHUNK, K), dst.reshape(NW, NCHUNK, K)
    )
    return _fin_call(parts, dis, b.reshape(1, CH))
